# baseline (speedup 1.0000x reference)
import functools

import jax
import jax.numpy as jnp
from jax import lax
from jax.experimental import pallas as pl
from jax.experimental.pallas import tpu as pltpu
from jax.experimental.pallas import tpu_sc as plsc

N_USERS = 25000
N_ITEMS = 25000
N_NODES = N_USERS + N_ITEMS
N_EDGES = 800000
D = 64

NC = 2
NS = 16
L = 16

HALF = N_NODES // NC
WPT = 1568
ACC_ROWS = WPT * NS
DUMP = HALF
W_LAST = HALF - 15 * WPT

CHUNK = 192
SUB = 96
NSUB = CHUNK // SUB
GROUPS = CHUNK // L

P_CHUNKS = 268
E_PER_TILE = P_CHUNKS * CHUNK
TOT_CHUNKS = NS * P_CHUNKS + 2
E_ALLOC = TOT_CHUNKS * CHUNK

N_CHUNKS = 140
CAP = N_CHUNKS * CHUNK
SEG = CAP + 2 * CHUNK
CUR_MAX = SEG - L
OUTER = N_CHUNKS // 4


def _part_kernel(edata_hbm, rows2_hbm, cols2_hbm, vals2_hbm,
                 ebuf, seg_r, seg_c, seg_v, sem_e):
    c = lax.axis_index("c")
    s = lax.axis_index("s")
    base_half = c * HALF
    tile_chunk0 = s * P_CHUNKS

    def fire(i, jj):
        pltpu.async_copy(edata_hbm.at[tile_chunk0 + i], ebuf.at[jj], sem_e)

    def wait(jj):
        pltpu.make_async_copy(edata_hbm.at[0], ebuf.at[jj], sem_e).wait()

    pad_r = jnp.broadcast_to(base_half, (L,)).astype(jnp.int32)
    pad_c = jnp.zeros((L,), jnp.int32)
    pad_v = jnp.zeros((L,), jnp.float32)

    def prefill(t, _):
        o = t * L
        seg_r[pl.ds(o, L)] = pad_r
        seg_c[pl.ds(o, L)] = pad_c
        seg_v[pl.ds(o, L)] = pad_v
        return 0
    lax.fori_loop(0, SEG // L, prefill, 0)

    fire(0, 0)
    fire(1, 1)

    def chunk_body(i, cur):
        jb = i % 2
        wait(jb)
        for m in range(GROUPS):
            r16 = ebuf[jb, pl.ds(m * L, L)]
            c16 = ebuf[jb, pl.ds(CHUNK + m * L, L)]
            v16 = ebuf[jb, pl.ds(2 * CHUNK + m * L, L)]
            l16 = r16 - base_half
            keep = (l16 >= 0) & (l16 < HALF)
            plsc.store_compressed(seg_r.at[pl.ds(cur, L)], r16, mask=keep)
            plsc.store_compressed(seg_c.at[pl.ds(cur, L)], c16, mask=keep)
            plsc.store_compressed(seg_v.at[pl.ds(cur, L)],
                                  plsc.bitcast(v16, jnp.float32), mask=keep)
            cnt = plsc.all_reduce_population_count(keep)
            cur = jnp.minimum(cur + cnt[0], CUR_MAX)
        fire(i + 2, jb)
        return cur

    lax.fori_loop(0, P_CHUNKS, chunk_body, jnp.int32(0))
    wait(P_CHUNKS % 2)
    wait((P_CHUNKS + 1) % 2)

    sbase = (c * NS + s) * SEG
    pltpu.sync_copy(seg_r, rows2_hbm.at[pl.ds(sbase, SEG)])
    pltpu.sync_copy(seg_c, cols2_hbm.at[pl.ds(sbase, SEG)])
    pltpu.sync_copy(seg_v, vals2_hbm.at[pl.ds(sbase, SEG)])


def _hop_kernel(x_hbm, rows2_hbm, cols2_hbm, vals2_hbm, y_hbm,
                acc, gbuf, rows4, cols4, vals4, lidx,
                sem_e, sem_g, sem_s):
    c = lax.axis_index("c")
    s = lax.axis_index("s")
    base_half = c * HALF

    seg_base = (c * NS + s) * SEG

    def fire_edges(i, jj):
        eb = seg_base + i * CHUNK
        pltpu.async_copy(rows2_hbm.at[pl.ds(eb, CHUNK)], rows4.at[jj], sem_e)
        pltpu.async_copy(cols2_hbm.at[pl.ds(eb, CHUNK)], cols4.at[jj], sem_e)
        pltpu.async_copy(vals2_hbm.at[pl.ds(eb, CHUNK)], vals4.at[jj], sem_e)

    def wait_edges(jj):
        pltpu.make_async_copy(rows2_hbm.at[pl.ds(0, CHUNK)],
                              rows4.at[jj], sem_e).wait()
        pltpu.make_async_copy(cols2_hbm.at[pl.ds(0, CHUNK)],
                              cols4.at[jj], sem_e).wait()
        pltpu.make_async_copy(vals2_hbm.at[pl.ds(0, CHUNK)],
                              vals4.at[jj], sem_e).wait()

    def fire_gather(jj, b):
        for k in range(NSUB):
            pltpu.async_copy(
                x_hbm.at[cols4.at[jj, pl.ds(k * SUB, SUB)]],
                gbuf.at[b, pl.ds(k * SUB, SUB)], sem_g)

    def wait_gather(jj, b):
        for k in range(NSUB):
            pltpu.make_async_copy(
                x_hbm.at[cols4.at[jj, pl.ds(k * SUB, SUB)]],
                gbuf.at[b, pl.ds(k * SUB, SUB)], sem_g).wait()

    def fire_scatter(b):
        for k in range(NSUB):
            pltpu.async_copy(gbuf.at[b, pl.ds(k * SUB, SUB)],
                             acc.at[lidx.at[b, k]], sem_s, add=True)

    def wait_scatter(b):
        for k in range(NSUB):
            pltpu.make_async_copy(gbuf.at[b, pl.ds(k * SUB, SUB)],
                                  acc.at[lidx.at[b, k]], sem_s).wait()

    fire_edges(0, 0)
    fire_edges(1, 1)

    def zrow(r, _):
        for q in range(D // L):
            gbuf[0, r, pl.ds(q * L, L)] = jnp.zeros((L,), jnp.float32)
        return 0
    lax.fori_loop(0, CHUNK, zrow, 0)
    dump16 = jnp.full((L,), DUMP, jnp.int32)
    for b in range(2):
        for k in range(NSUB):
            for m in range(SUB // L):
                lidx[b, k, pl.ds(m * L, L)] = dump16

    zstart = s * WPT
    for k in range(WPT // CHUNK):
        pltpu.sync_copy(gbuf.at[0], acc.at[pl.ds(zstart + k * CHUNK, CHUNK)])
    zrem = WPT % CHUNK
    pltpu.sync_copy(gbuf.at[0, pl.ds(0, zrem)],
                    acc.at[pl.ds(zstart + (WPT // CHUNK) * CHUNK, zrem)])
    plsc.subcore_barrier()

    for k in range(NSUB):
        pltpu.async_copy(gbuf.at[0, pl.ds(k * SUB, SUB)],
                         acc.at[lidx.at[1, k]], sem_s, add=True)
    wait_edges(0)
    fire_gather(0, 0)

    def outer(g, _):
        for j in range(4):
            i = g * 4 + j
            b = j % 2
            wait_edges((j + 1) % 4)
            fire_edges(i + 2, (j + 2) % 4)
            for m in range(GROUPS):
                r16 = rows4[j, pl.ds(m * L, L)]
                l16 = r16 - base_half
                oob = (l16 < 0) | (l16 >= HALF)
                l16 = jnp.where(oob, DUMP, l16)
                lidx[b, m // (SUB // L), pl.ds((m % (SUB // L)) * L, L)] = l16
            wait_gather(j, b)
            def scale16(m, _):
                r0 = m * L
                v16 = vals4[j, pl.ds(r0, L)]
                for ii in range(L):
                    v = v16[ii]
                    for q in range(D // L):
                        gbuf[b, r0 + ii, pl.ds(q * L, L)] = (
                            gbuf[b, r0 + ii, pl.ds(q * L, L)] * v)
                return 0
            lax.fori_loop(0, GROUPS, scale16, 0)
            wait_scatter(1 - b)
            fire_gather((j + 1) % 4, 1 - b)
            fire_scatter(b)
        return 0

    lax.fori_loop(0, OUTER, outer, 0)

    wait_edges((N_CHUNKS + 1) % 4)
    wait_gather(N_CHUNKS % 4, N_CHUNKS % 2)
    wait_scatter((N_CHUNKS - 1) % 2)
    plsc.subcore_barrier()

    wstart = s * WPT

    @pl.when(s < NS - 1)
    def _():
        for k in range(WPT // CHUNK):
            off = wstart + k * CHUNK
            pltpu.sync_copy(acc.at[pl.ds(off, CHUNK)],
                            y_hbm.at[pl.ds(base_half + off, CHUNK)])
        toff = wstart + (WPT // CHUNK) * CHUNK
        pltpu.sync_copy(acc.at[pl.ds(toff, WPT % CHUNK)],
                        y_hbm.at[pl.ds(base_half + toff, WPT % CHUNK)])

    @pl.when(s == NS - 1)
    def _():
        for k in range(W_LAST // CHUNK):
            off = wstart + k * CHUNK
            pltpu.sync_copy(acc.at[pl.ds(off, CHUNK)],
                            y_hbm.at[pl.ds(base_half + off, CHUNK)])
        toff = wstart + (W_LAST // CHUNK) * CHUNK
        pltpu.sync_copy(acc.at[pl.ds(toff, W_LAST % CHUNK)],
                        y_hbm.at[pl.ds(base_half + toff, W_LAST % CHUNK)])


_mesh = plsc.VectorSubcoreMesh(core_axis_name="c", subcore_axis_name="s")

_part = functools.partial(
    pl.kernel,
    mesh=_mesh,
    compiler_params=pltpu.CompilerParams(use_tc_tiling_on_sc=False,
                                         needs_layout_passes=False),
    out_type=(jax.ShapeDtypeStruct((NC * NS * SEG,), jnp.int32),
              jax.ShapeDtypeStruct((NC * NS * SEG,), jnp.int32),
              jax.ShapeDtypeStruct((NC * NS * SEG,), jnp.float32)),
    scratch_types=[
        pltpu.VMEM((2, 3 * CHUNK), jnp.int32),
        pltpu.VMEM((SEG,), jnp.int32),
        pltpu.VMEM((SEG,), jnp.int32),
        pltpu.VMEM((SEG,), jnp.float32),
        pltpu.SemaphoreType.DMA,
    ],
)(_part_kernel)

_hop = functools.partial(
    pl.kernel,
    mesh=_mesh,
    compiler_params=pltpu.CompilerParams(use_tc_tiling_on_sc=False,
                                         needs_layout_passes=False),
    out_type=jax.ShapeDtypeStruct((N_NODES, D), jnp.float32),
    scratch_types=[
        pltpu.VMEM_SHARED((ACC_ROWS, D), jnp.float32),
        pltpu.VMEM((2, CHUNK, D), jnp.float32),
        pltpu.VMEM((4, CHUNK), jnp.int32),
        pltpu.VMEM((4, CHUNK), jnp.int32),
        pltpu.VMEM((4, CHUNK), jnp.float32),
        pltpu.VMEM((2, NSUB, SUB), jnp.int32),
        pltpu.SemaphoreType.DMA,
        pltpu.SemaphoreType.DMA,
        pltpu.SemaphoreType.DMA,
    ],
)(_hop_kernel)


def kernel(user_embed, item_embed, edge_rows, edge_cols, edge_vals):
    x0 = jnp.concatenate([user_embed, item_embed], axis=0)
    pad = E_ALLOC - N_EDGES
    rows_p = jnp.concatenate([edge_rows, jnp.full((pad,), N_NODES, jnp.int32)])
    cols_p = jnp.concatenate([edge_cols, jnp.zeros((pad,), jnp.int32)])
    vals_p = jnp.concatenate([edge_vals, jnp.zeros((pad,), jnp.float32)])
    edata = jnp.concatenate(
        [rows_p.reshape(TOT_CHUNKS, CHUNK),
         cols_p.reshape(TOT_CHUNKS, CHUNK),
         jax.lax.bitcast_convert_type(vals_p, jnp.int32)
             .reshape(TOT_CHUNKS, CHUNK)],
        axis=1)

    rows2, cols2, vals2 = _part(edata)
    x1 = _hop(x0, rows2, cols2, vals2)
    x2 = _hop(x1, rows2, cols2, vals2)
    x3 = _hop(x2, rows2, cols2, vals2)
    embs = jnp.stack([x0, x1, x2, x3], axis=1)
    return (embs[:N_USERS], embs[N_USERS:])


# speedup vs baseline: 3.5662x; 3.5662x over previous
import functools

import jax
import jax.numpy as jnp
from jax import lax
from jax.experimental import pallas as pl
from jax.experimental.pallas import tpu as pltpu
from jax.experimental.pallas import tpu_sc as plsc

N_USERS = 25000
N_ITEMS = 25000
N_NODES = N_USERS + N_ITEMS
N_EDGES = 800000
D = 64

NC = 2
NS = 16
L = 16

HALF = N_NODES // NC
WPT = 1568
ACC_ROWS = WPT * NS
DUMP = HALF
W_LAST = HALF - 15 * WPT

CHUNK = 192
SUB = 96
NSUB = CHUNK // SUB
GROUPS = CHUNK // L

P_CHUNKS = 268
E_PER_TILE = P_CHUNKS * CHUNK
TOT_CHUNKS = NS * P_CHUNKS + 2
E_ALLOC = TOT_CHUNKS * CHUNK

N_CHUNKS = 140
CAP = N_CHUNKS * CHUNK
SEG = CAP + 2 * CHUNK
CUR_MAX = SEG - L
OUTER = N_CHUNKS // 4


def _part_kernel(edata_hbm, rows2_hbm, cols2_hbm, vals2_hbm,
                 ebuf, seg_r, seg_c, seg_v, sem_e):
    c = lax.axis_index("c")
    s = lax.axis_index("s")
    base_half = c * HALF
    tile_chunk0 = s * P_CHUNKS

    def fire(i, jj):
        pltpu.async_copy(edata_hbm.at[tile_chunk0 + i], ebuf.at[jj], sem_e)

    def wait(jj):
        pltpu.make_async_copy(edata_hbm.at[0], ebuf.at[jj], sem_e).wait()

    pad_r = jnp.broadcast_to(base_half, (L,)).astype(jnp.int32)
    iota16 = lax.iota(jnp.int32, L)
    pad_v = jnp.zeros((L,), jnp.float32)

    def prefill(t, _):
        o = t * L
        seg_r[pl.ds(o, L)] = pad_r
        seg_c[pl.ds(o, L)] = iota16 + o
        seg_v[pl.ds(o, L)] = pad_v
        return 0
    lax.fori_loop(0, SEG // L, prefill, 0)

    fire(0, 0)
    fire(1, 1)

    def chunk_body(i, cur):
        jb = i % 2
        wait(jb)
        for m in range(GROUPS):
            r16 = ebuf[jb, pl.ds(m * L, L)]
            c16 = ebuf[jb, pl.ds(CHUNK + m * L, L)]
            v16 = ebuf[jb, pl.ds(2 * CHUNK + m * L, L)]
            l16 = r16 - base_half
            keep = (l16 >= 0) & (l16 < HALF)
            plsc.store_compressed(seg_r.at[pl.ds(cur, L)], r16, mask=keep)
            plsc.store_compressed(seg_c.at[pl.ds(cur, L)], c16, mask=keep)
            plsc.store_compressed(seg_v.at[pl.ds(cur, L)],
                                  plsc.bitcast(v16, jnp.float32), mask=keep)
            cnt = plsc.all_reduce_population_count(keep)
            cur = jnp.minimum(cur + cnt[0], CUR_MAX)
        fire(i + 2, jb)
        return cur

    lax.fori_loop(0, P_CHUNKS, chunk_body, jnp.int32(0))
    wait(P_CHUNKS % 2)
    wait((P_CHUNKS + 1) % 2)

    sbase = (c * NS + s) * SEG
    pltpu.sync_copy(seg_r, rows2_hbm.at[pl.ds(sbase, SEG)])
    pltpu.sync_copy(seg_c, cols2_hbm.at[pl.ds(sbase, SEG)])
    pltpu.sync_copy(seg_v, vals2_hbm.at[pl.ds(sbase, SEG)])


def _hop_kernel(x_hbm, rows2_hbm, cols2_hbm, vals2_hbm, y_hbm,
                acc, gbuf, rows4, cols4, vals4, lidx,
                sem_e, sem_g, sem_s):
    c = lax.axis_index("c")
    s = lax.axis_index("s")
    base_half = c * HALF

    seg_base = (c * NS + s) * SEG

    def fire_edges(i, jj):
        eb = seg_base + i * CHUNK
        pltpu.async_copy(rows2_hbm.at[pl.ds(eb, CHUNK)], rows4.at[jj], sem_e)
        pltpu.async_copy(cols2_hbm.at[pl.ds(eb, CHUNK)], cols4.at[jj], sem_e)
        pltpu.async_copy(vals2_hbm.at[pl.ds(eb, CHUNK)], vals4.at[jj], sem_e)

    def wait_edges(jj):
        pltpu.make_async_copy(rows2_hbm.at[pl.ds(0, CHUNK)],
                              rows4.at[jj], sem_e).wait()
        pltpu.make_async_copy(cols2_hbm.at[pl.ds(0, CHUNK)],
                              cols4.at[jj], sem_e).wait()
        pltpu.make_async_copy(vals2_hbm.at[pl.ds(0, CHUNK)],
                              vals4.at[jj], sem_e).wait()

    def fire_gather(jj, b):
        for k in range(NSUB):
            pltpu.async_copy(
                x_hbm.at[cols4.at[jj, pl.ds(k * SUB, SUB)]],
                gbuf.at[b, pl.ds(k * SUB, SUB)], sem_g)

    def wait_gather(jj, b):
        for k in range(NSUB):
            pltpu.make_async_copy(
                x_hbm.at[cols4.at[jj, pl.ds(k * SUB, SUB)]],
                gbuf.at[b, pl.ds(k * SUB, SUB)], sem_g).wait()

    def fire_scatter(b):
        for k in range(NSUB):
            pltpu.async_copy(gbuf.at[b, pl.ds(k * SUB, SUB)],
                             acc.at[lidx.at[b, k]], sem_s, add=True)

    def wait_scatter(b):
        for k in range(NSUB):
            pltpu.make_async_copy(gbuf.at[b, pl.ds(k * SUB, SUB)],
                                  acc.at[lidx.at[b, k]], sem_s).wait()

    fire_edges(0, 0)
    fire_edges(1, 1)

    def zrow(r, _):
        for q in range(D // L):
            gbuf[0, r, pl.ds(q * L, L)] = jnp.zeros((L,), jnp.float32)
        return 0
    lax.fori_loop(0, CHUNK, zrow, 0)
    dump16 = jnp.full((L,), DUMP, jnp.int32)
    for b in range(2):
        for k in range(NSUB):
            for m in range(SUB // L):
                lidx[b, k, pl.ds(m * L, L)] = dump16

    zstart = s * WPT
    for k in range(WPT // CHUNK):
        pltpu.sync_copy(gbuf.at[0], acc.at[pl.ds(zstart + k * CHUNK, CHUNK)])
    zrem = WPT % CHUNK
    pltpu.sync_copy(gbuf.at[0, pl.ds(0, zrem)],
                    acc.at[pl.ds(zstart + (WPT // CHUNK) * CHUNK, zrem)])
    plsc.subcore_barrier()

    for k in range(NSUB):
        pltpu.async_copy(gbuf.at[0, pl.ds(k * SUB, SUB)],
                         acc.at[lidx.at[1, k]], sem_s, add=True)
    wait_edges(0)
    fire_gather(0, 0)

    def outer(g, _):
        for j in range(4):
            i = g * 4 + j
            b = j % 2
            wait_edges((j + 1) % 4)
            fire_edges(i + 2, (j + 2) % 4)
            for m in range(GROUPS):
                r16 = rows4[j, pl.ds(m * L, L)]
                l16 = r16 - base_half
                oob = (l16 < 0) | (l16 >= HALF)
                l16 = jnp.where(oob, DUMP, l16)
                lidx[b, m // (SUB // L), pl.ds((m % (SUB // L)) * L, L)] = l16
            wait_gather(j, b)
            def scale16(m, _):
                r0 = m * L
                v16 = vals4[j, pl.ds(r0, L)]
                for ii in range(L):
                    v = v16[ii]
                    for q in range(D // L):
                        gbuf[b, r0 + ii, pl.ds(q * L, L)] = (
                            gbuf[b, r0 + ii, pl.ds(q * L, L)] * v)
                return 0
            lax.fori_loop(0, GROUPS, scale16, 0)
            wait_scatter(1 - b)
            fire_gather((j + 1) % 4, 1 - b)
            fire_scatter(b)
        return 0

    lax.fori_loop(0, OUTER, outer, 0)

    wait_edges((N_CHUNKS + 1) % 4)
    wait_gather(N_CHUNKS % 4, N_CHUNKS % 2)
    wait_scatter((N_CHUNKS - 1) % 2)
    plsc.subcore_barrier()

    wstart = s * WPT

    @pl.when(s < NS - 1)
    def _():
        for k in range(WPT // CHUNK):
            off = wstart + k * CHUNK
            pltpu.sync_copy(acc.at[pl.ds(off, CHUNK)],
                            y_hbm.at[pl.ds(base_half + off, CHUNK)])
        toff = wstart + (WPT // CHUNK) * CHUNK
        pltpu.sync_copy(acc.at[pl.ds(toff, WPT % CHUNK)],
                        y_hbm.at[pl.ds(base_half + toff, WPT % CHUNK)])

    @pl.when(s == NS - 1)
    def _():
        for k in range(W_LAST // CHUNK):
            off = wstart + k * CHUNK
            pltpu.sync_copy(acc.at[pl.ds(off, CHUNK)],
                            y_hbm.at[pl.ds(base_half + off, CHUNK)])
        toff = wstart + (W_LAST // CHUNK) * CHUNK
        pltpu.sync_copy(acc.at[pl.ds(toff, W_LAST % CHUNK)],
                        y_hbm.at[pl.ds(base_half + toff, W_LAST % CHUNK)])


_mesh = plsc.VectorSubcoreMesh(core_axis_name="c", subcore_axis_name="s")

_part = functools.partial(
    pl.kernel,
    mesh=_mesh,
    compiler_params=pltpu.CompilerParams(use_tc_tiling_on_sc=False,
                                         needs_layout_passes=False),
    out_type=(jax.ShapeDtypeStruct((NC * NS * SEG,), jnp.int32),
              jax.ShapeDtypeStruct((NC * NS * SEG,), jnp.int32),
              jax.ShapeDtypeStruct((NC * NS * SEG,), jnp.float32)),
    scratch_types=[
        pltpu.VMEM((2, 3 * CHUNK), jnp.int32),
        pltpu.VMEM((SEG,), jnp.int32),
        pltpu.VMEM((SEG,), jnp.int32),
        pltpu.VMEM((SEG,), jnp.float32),
        pltpu.SemaphoreType.DMA,
    ],
)(_part_kernel)

_hop = functools.partial(
    pl.kernel,
    mesh=_mesh,
    compiler_params=pltpu.CompilerParams(use_tc_tiling_on_sc=False,
                                         needs_layout_passes=False),
    out_type=jax.ShapeDtypeStruct((N_NODES, D), jnp.float32),
    scratch_types=[
        pltpu.VMEM_SHARED((ACC_ROWS, D), jnp.float32),
        pltpu.VMEM((2, CHUNK, D), jnp.float32),
        pltpu.VMEM((4, CHUNK), jnp.int32),
        pltpu.VMEM((4, CHUNK), jnp.int32),
        pltpu.VMEM((4, CHUNK), jnp.float32),
        pltpu.VMEM((2, NSUB, SUB), jnp.int32),
        pltpu.SemaphoreType.DMA,
        pltpu.SemaphoreType.DMA,
        pltpu.SemaphoreType.DMA,
    ],
)(_hop_kernel)


def kernel(user_embed, item_embed, edge_rows, edge_cols, edge_vals):
    x0 = jnp.concatenate([user_embed, item_embed], axis=0)
    pad = E_ALLOC - N_EDGES
    rows_p = jnp.concatenate([edge_rows, jnp.full((pad,), N_NODES, jnp.int32)])
    cols_p = jnp.concatenate([edge_cols, jnp.zeros((pad,), jnp.int32)])
    vals_p = jnp.concatenate([edge_vals, jnp.zeros((pad,), jnp.float32)])
    edata = jnp.concatenate(
        [rows_p.reshape(TOT_CHUNKS, CHUNK),
         cols_p.reshape(TOT_CHUNKS, CHUNK),
         jax.lax.bitcast_convert_type(vals_p, jnp.int32)
             .reshape(TOT_CHUNKS, CHUNK)],
        axis=1)

    rows2, cols2, vals2 = _part(edata)
    x1 = _hop(x0, rows2, cols2, vals2)
    x2 = _hop(x1, rows2, cols2, vals2)
    x3 = _hop(x2, rows2, cols2, vals2)
    embs = jnp.stack([x0, x1, x2, x3], axis=1)
    return (embs[:N_USERS], embs[N_USERS:])
